# Initial kernel scaffold; baseline (speedup 1.0000x reference)
import jax
import jax.numpy as jnp
from jax.experimental import pallas as pl


def kernel(x, edge_index, W_src1, W_dst1, a_src1, a_dst1, b1, W_src2, W_dst2, a_src2, a_dst2, b2, W_lin, b_lin):
    raise NotImplementedError("write your pallas kernel here")


# speedup vs baseline: 9.1168x; 9.1168x over previous
import functools

import jax
import jax.numpy as jnp
from jax import lax
from jax.experimental import pallas as pl
from jax.experimental.pallas import tpu as pltpu
from jax.experimental.pallas import tpu_sc as plsc

N = 10000
D = 128
H = 128
E = 320000

NC = 2
NS = 16
LANES = 16
NW = NC * NS

ESTEP = 32
GROUPS = ESTEP // LANES
STEPS = 320
PER_W = STEPS * ESTEP
E_PAD = PER_W * NW
N_ACC = 10240
DROWS = N_ACC // 128
ZROWS = N_ACC // NS

_mesh = plsc.VectorSubcoreMesh(core_axis_name="c", subcore_axis_name="s")


@functools.partial(
    pl.kernel,
    out_type=[
        jax.ShapeDtypeStruct((NC, N_ACC, H), jnp.float32),
        jax.ShapeDtypeStruct((NC, DROWS, 128), jnp.float32),
    ],
    mesh=_mesh,
    scratch_types=[
        pltpu.VMEM((N,), jnp.float32),
        pltpu.VMEM((N + LANES,), jnp.float32),
        pltpu.VMEM((ESTEP,), jnp.int32),
        pltpu.VMEM((ESTEP,), jnp.int32),
        pltpu.VMEM((ESTEP,), jnp.int32),
        pltpu.VMEM((ESTEP, H), jnp.float32),
        pltpu.VMEM((ESTEP, 128), jnp.float32),
        pltpu.VMEM_SHARED((N_ACC, H), jnp.float32),
        pltpu.VMEM_SHARED((DROWS, 128), jnp.float32),
        pltpu.SemaphoreType.DMA,
    ],
    compiler_params=pltpu.CompilerParams(needs_layout_passes=False),
)
def _edge_pass(h_hbm, as_hbm, ad_hbm, src_hbm, dst_hbm, z1_hbm, z2_hbm,
               num_hbm, den_hbm,
               as_v, ad_v, src_r, dst_r, dsh_r, rows_v, oh_v,
               acc_n, acc_d, sem):
    c = lax.axis_index("c")
    s = lax.axis_index("s")
    wid = s * NC + c

    pltpu.sync_copy(as_hbm, as_v)
    pltpu.sync_copy(ad_hbm, ad_v.at[pl.ds(0, N)])
    ad_v[pl.ds(N, LANES)] = jnp.zeros((LANES,), jnp.float32)

    zbase = pl.multiple_of(s * ZROWS, 8)
    pltpu.sync_copy(z1_hbm, acc_n.at[pl.ds(zbase, ZROWS)])

    @pl.when(s == 0)
    def _zero_den():
        pltpu.sync_copy(z2_hbm, acc_d)
    plsc.subcore_barrier()

    lane_iota = lax.iota(jnp.int32, LANES)
    ebase0 = pl.multiple_of(wid * PER_W, 8)

    def _step(j, carry):
        ebase = pl.multiple_of(ebase0 + j * ESTEP, 8)
        pltpu.sync_copy(src_hbm.at[pl.ds(ebase, ESTEP)], src_r)
        pltpu.sync_copy(dst_hbm.at[pl.ds(ebase, ESTEP)], dst_r)
        pltpu.async_copy(h_hbm.at[src_r], rows_v, sem).wait()
        for g in range(GROUPS):
            sidx = src_r[pl.ds(g * LANES, LANES)]
            didx = dst_r[pl.ds(g * LANES, LANES)]
            a_s = plsc.load_gather(as_v, [sidx])
            a_d = plsc.load_gather(ad_v, [didx])
            e = a_s + a_d
            e = jnp.where(e > 0, e, 0.2 * e)
            ex = jnp.exp(e)
            dsh_r[pl.ds(g * LANES, LANES)] = lax.shift_right_logical(didx, 7)
            dlow = jnp.bitwise_and(didx, 127)
            for l in range(LANES):
                r = g * LANES + l
                lsel = jnp.full((LANES,), l, jnp.int32)
                sc = ex.at[lsel].get(mode='promise_in_bounds')
                dl = dlow.at[lsel].get(mode='promise_in_bounds')
                for q in range(128 // LANES):
                    oh_v[r, pl.ds(q * LANES, LANES)] = jnp.where(
                        lane_iota + q * LANES == dl, sc, 0.0)
                for q in range(H // LANES):
                    rows_v[r, pl.ds(q * LANES, LANES)] = (
                        rows_v[r, pl.ds(q * LANES, LANES)] * sc)
        pltpu.sync_copy(rows_v, acc_n.at[dst_r], add=True)
        pltpu.sync_copy(oh_v, acc_d.at[dsh_r], add=True)
        return carry

    lax.fori_loop(0, STEPS, _step, jnp.int32(0))
    plsc.subcore_barrier()

    pltpu.sync_copy(acc_n.at[pl.ds(zbase, ZROWS)],
                    num_hbm.at[c].at[pl.ds(zbase, ZROWS)])

    @pl.when(s == 0)
    def _wb_den():
        pltpu.sync_copy(acc_d, den_hbm.at[c])


def _leaky(v):
    return jnp.where(v > 0, v, 0.2 * v)


def _prep_body(x_ref, ws_ref, wd_ref, avs_ref, avd_ref,
               h_ref, as_ref, ad_ref, m_ref):
    xv = x_ref[...]
    h = jnp.dot(xv, ws_ref[...], preferred_element_type=jnp.float32)
    hd = jnp.dot(xv, wd_ref[...], preferred_element_type=jnp.float32)
    a_s = jnp.dot(h, avs_ref[...], preferred_element_type=jnp.float32)
    a_d = jnp.dot(hd, avd_ref[...], preferred_element_type=jnp.float32)
    h_ref[...] = h
    as_ref[...] = a_s
    ad_ref[...] = a_d
    mval = _leaky(jnp.max(a_s) + jnp.max(a_d))
    m_ref[...] = jnp.full((1, LANES), mval, jnp.float32)


_prep = pl.pallas_call(
    _prep_body,
    out_shape=[
        jax.ShapeDtypeStruct((N, H), jnp.float32),
        jax.ShapeDtypeStruct((N, 1), jnp.float32),
        jax.ShapeDtypeStruct((N, 1), jnp.float32),
        jax.ShapeDtypeStruct((1, LANES), jnp.float32),
    ],
)


def _combine_prep_body(pn_ref, pd_ref, b_ref, ws_ref, wd_ref, avs_ref, avd_ref,
                       h_ref, as_ref, ad_ref, m_ref):
    num = pn_ref[0, :N, :] + pn_ref[1, :N, :]
    den = pd_ref[0, :N, :] + pd_ref[1, :N, :]
    xl = num / (den + 1e-16) + b_ref[...]
    xl = jnp.where(xl > 0, xl, jnp.exp(xl) - 1.0)
    h = jnp.dot(xl, ws_ref[...], preferred_element_type=jnp.float32)
    hd = jnp.dot(xl, wd_ref[...], preferred_element_type=jnp.float32)
    a_s = jnp.dot(h, avs_ref[...], preferred_element_type=jnp.float32)
    a_d = jnp.dot(hd, avd_ref[...], preferred_element_type=jnp.float32)
    h_ref[...] = h
    as_ref[...] = a_s
    ad_ref[...] = a_d
    mval = _leaky(jnp.max(a_s) + jnp.max(a_d))
    m_ref[...] = jnp.full((1, LANES), mval, jnp.float32)


_combine_prep = pl.pallas_call(
    _combine_prep_body,
    out_shape=[
        jax.ShapeDtypeStruct((N, H), jnp.float32),
        jax.ShapeDtypeStruct((N, 1), jnp.float32),
        jax.ShapeDtypeStruct((N, 1), jnp.float32),
        jax.ShapeDtypeStruct((1, LANES), jnp.float32),
    ],
)


def _final_body(pn_ref, pd_ref, b_ref, wl_ref, bl_ref, out_ref):
    num = pn_ref[0, :N, :] + pn_ref[1, :N, :]
    den = pd_ref[0, :N, :] + pd_ref[1, :N, :]
    xl = num / (den + 1e-16) + b_ref[...]
    xl = jnp.where(xl > 0, xl, jnp.exp(xl) - 1.0)
    out_ref[...] = jnp.dot(xl, wl_ref[...],
                           preferred_element_type=jnp.float32) + bl_ref[...]


_final = pl.pallas_call(
    _final_body,
    out_shape=jax.ShapeDtypeStruct((N, 1), jnp.float32),
)


def kernel(x, edge_index, W_src1, W_dst1, a_src1, a_dst1, b1,
           W_src2, W_dst2, a_src2, a_dst2, b2, W_lin, b_lin):
    src = edge_index[0]
    dst = edge_index[1]
    pad = E_PAD - E
    srcp = jnp.concatenate([src, jnp.zeros((pad,), jnp.int32)])
    dstp = jnp.concatenate([dst, jnp.full((pad,), N, jnp.int32)])
    z1 = jnp.zeros((ZROWS, H), jnp.float32)
    z2 = jnp.zeros((DROWS, 128), jnp.float32)

    h1, as1, ad1, _m1 = _prep(x, W_src1, W_dst1,
                              a_src1.reshape(H, 1), a_dst1.reshape(H, 1))
    pn1, pd1 = _edge_pass(h1, as1.reshape(N), ad1.reshape(N),
                          srcp, dstp, z1, z2)
    pd1 = pd1.reshape(NC, N_ACC, 1)
    h2, as2, ad2, _m2 = _combine_prep(pn1, pd1, b1.reshape(1, H),
                                      W_src2, W_dst2,
                                      a_src2.reshape(H, 1), a_dst2.reshape(H, 1))
    pn2, pd2 = _edge_pass(h2, as2.reshape(N), ad2.reshape(N),
                          srcp, dstp, z1, z2)
    pd2 = pd2.reshape(NC, N_ACC, 1)
    out = _final(pn2, pd2, b2.reshape(1, H), W_lin, b_lin.reshape(1, 1))
    return out.reshape(N)


# speedup vs baseline: 15.6417x; 1.7157x over previous
import functools

import jax
import jax.numpy as jnp
from jax import lax
from jax.experimental import pallas as pl
from jax.experimental.pallas import tpu as pltpu
from jax.experimental.pallas import tpu_sc as plsc

N = 10000
D = 128
H = 128
E = 320000

NC = 2
NS = 16
LANES = 16
NW = NC * NS

ESTEP = 32
GROUPS = ESTEP // LANES
STEPS = 320
PER_W = STEPS * ESTEP
E_PAD = PER_W * NW
N_ACC = 10240
DROWS = N_ACC // 128
ZROWS = N_ACC // NS

_mesh = plsc.VectorSubcoreMesh(core_axis_name="c", subcore_axis_name="s")


@functools.partial(
    pl.kernel,
    out_type=[
        jax.ShapeDtypeStruct((NC, N_ACC, H), jnp.float32),
        jax.ShapeDtypeStruct((NC, DROWS, 128), jnp.float32),
    ],
    mesh=_mesh,
    scratch_types=[
        pltpu.VMEM((N,), jnp.float32),
        pltpu.VMEM((N + LANES,), jnp.float32),
        pltpu.VMEM((ESTEP,), jnp.int32),
        pltpu.VMEM((ESTEP,), jnp.int32),
        pltpu.VMEM((ESTEP,), jnp.int32),
        pltpu.VMEM((ESTEP,), jnp.int32),
        pltpu.VMEM((ESTEP,), jnp.int32),
        pltpu.VMEM((ESTEP, H), jnp.float32),
        pltpu.VMEM((ESTEP, H), jnp.float32),
        pltpu.VMEM((ESTEP, 128), jnp.float32),
        pltpu.VMEM_SHARED((N_ACC, H), jnp.float32),
        pltpu.VMEM_SHARED((DROWS, 128), jnp.float32),
        pltpu.SemaphoreType.DMA,
        pltpu.SemaphoreType.DMA,
    ],
    compiler_params=pltpu.CompilerParams(needs_layout_passes=False),
)
def _edge_pass(h_hbm, as_hbm, ad_hbm, src_hbm, dst_hbm, z1_hbm, z2_hbm,
               num_hbm, den_hbm,
               as_v, ad_v, src_a, dst_a, src_b, dst_b, dsh_r,
               rows_a, rows_b, oh_v,
               acc_n, acc_d, sem_a, sem_b):
    c = lax.axis_index("c")
    s = lax.axis_index("s")
    wid = s * NC + c

    pltpu.sync_copy(as_hbm, as_v)
    pltpu.sync_copy(ad_hbm, ad_v.at[pl.ds(0, N)])
    ad_v[pl.ds(N, LANES)] = jnp.zeros((LANES,), jnp.float32)

    zbase = pl.multiple_of(s * ZROWS, 8)
    pltpu.sync_copy(z1_hbm, acc_n.at[pl.ds(zbase, ZROWS)])

    @pl.when(s == 0)
    def _zero_den():
        pltpu.sync_copy(z2_hbm, acc_d)
    plsc.subcore_barrier()

    lane_iota = lax.iota(jnp.int32, LANES)
    ebase0 = pl.multiple_of(wid * PER_W, 8)

    def _compute(src_r, dst_r, rows_v):
        for g in range(GROUPS):
            sidx = src_r[pl.ds(g * LANES, LANES)]
            didx = dst_r[pl.ds(g * LANES, LANES)]
            a_s = plsc.load_gather(as_v, [sidx])
            a_d = plsc.load_gather(ad_v, [didx])
            e = a_s + a_d
            e = jnp.where(e > 0, e, 0.2 * e)
            ex = jnp.exp(e)
            dsh_r[pl.ds(g * LANES, LANES)] = lax.shift_right_logical(didx, 7)
            dlow = jnp.bitwise_and(didx, 127)
            for l in range(LANES):
                r = g * LANES + l
                lsel = jnp.full((LANES,), l, jnp.int32)
                sc = ex.at[lsel].get(mode='promise_in_bounds')
                dl = dlow.at[lsel].get(mode='promise_in_bounds')
                for q in range(128 // LANES):
                    oh_v[r, pl.ds(q * LANES, LANES)] = jnp.where(
                        lane_iota + q * LANES == dl, sc, 0.0)
                for q in range(H // LANES):
                    rows_v[r, pl.ds(q * LANES, LANES)] = (
                        rows_v[r, pl.ds(q * LANES, LANES)] * sc)
        pltpu.sync_copy(rows_v, acc_n.at[dst_r], add=True)
        pltpu.sync_copy(oh_v, acc_d.at[dsh_r], add=True)

    def _stage(j, src_r, dst_r):
        ebase = pl.multiple_of(ebase0 + j * ESTEP, 8)
        pltpu.sync_copy(src_hbm.at[pl.ds(ebase, ESTEP)], src_r)
        pltpu.sync_copy(dst_hbm.at[pl.ds(ebase, ESTEP)], dst_r)

    _stage(0, src_a, dst_a)
    pltpu.async_copy(h_hbm.at[src_a], rows_a, sem_a)

    def _pair(i, carry):
        _stage(2 * i + 1, src_b, dst_b)
        pltpu.async_copy(h_hbm.at[src_b], rows_b, sem_b)
        pltpu.make_async_copy(h_hbm.at[src_a], rows_a, sem_a).wait()
        _compute(src_a, dst_a, rows_a)
        _stage(2 * i + 2, src_a, dst_a)
        pltpu.async_copy(h_hbm.at[src_a], rows_a, sem_a)
        pltpu.make_async_copy(h_hbm.at[src_b], rows_b, sem_b).wait()
        _compute(src_b, dst_b, rows_b)
        return carry

    lax.fori_loop(0, STEPS // 2, _pair, jnp.int32(0))
    pltpu.make_async_copy(h_hbm.at[src_a], rows_a, sem_a).wait()
    plsc.subcore_barrier()

    pltpu.sync_copy(acc_n.at[pl.ds(zbase, ZROWS)],
                    num_hbm.at[c].at[pl.ds(zbase, ZROWS)])

    @pl.when(s == 0)
    def _wb_den():
        pltpu.sync_copy(acc_d, den_hbm.at[c])


def _leaky(v):
    return jnp.where(v > 0, v, 0.2 * v)


def _prep_body(x_ref, ws_ref, wd_ref, avs_ref, avd_ref,
               h_ref, as_ref, ad_ref, m_ref):
    xv = x_ref[...]
    h = jnp.dot(xv, ws_ref[...], preferred_element_type=jnp.float32)
    hd = jnp.dot(xv, wd_ref[...], preferred_element_type=jnp.float32)
    a_s = jnp.dot(h, avs_ref[...], preferred_element_type=jnp.float32)
    a_d = jnp.dot(hd, avd_ref[...], preferred_element_type=jnp.float32)
    h_ref[...] = h
    as_ref[...] = a_s
    ad_ref[...] = a_d
    mval = _leaky(jnp.max(a_s) + jnp.max(a_d))
    m_ref[...] = jnp.full((1, LANES), mval, jnp.float32)


_prep = pl.pallas_call(
    _prep_body,
    out_shape=[
        jax.ShapeDtypeStruct((N, H), jnp.float32),
        jax.ShapeDtypeStruct((N, 1), jnp.float32),
        jax.ShapeDtypeStruct((N, 1), jnp.float32),
        jax.ShapeDtypeStruct((1, LANES), jnp.float32),
    ],
)


def _combine_prep_body(pn_ref, pd_ref, b_ref, ws_ref, wd_ref, avs_ref, avd_ref,
                       h_ref, as_ref, ad_ref, m_ref):
    num = pn_ref[0, :N, :] + pn_ref[1, :N, :]
    den = pd_ref[0, :N, :] + pd_ref[1, :N, :]
    xl = num / (den + 1e-16) + b_ref[...]
    xl = jnp.where(xl > 0, xl, jnp.exp(xl) - 1.0)
    h = jnp.dot(xl, ws_ref[...], preferred_element_type=jnp.float32)
    hd = jnp.dot(xl, wd_ref[...], preferred_element_type=jnp.float32)
    a_s = jnp.dot(h, avs_ref[...], preferred_element_type=jnp.float32)
    a_d = jnp.dot(hd, avd_ref[...], preferred_element_type=jnp.float32)
    h_ref[...] = h
    as_ref[...] = a_s
    ad_ref[...] = a_d
    mval = _leaky(jnp.max(a_s) + jnp.max(a_d))
    m_ref[...] = jnp.full((1, LANES), mval, jnp.float32)


_combine_prep = pl.pallas_call(
    _combine_prep_body,
    out_shape=[
        jax.ShapeDtypeStruct((N, H), jnp.float32),
        jax.ShapeDtypeStruct((N, 1), jnp.float32),
        jax.ShapeDtypeStruct((N, 1), jnp.float32),
        jax.ShapeDtypeStruct((1, LANES), jnp.float32),
    ],
)


def _final_body(pn_ref, pd_ref, b_ref, wl_ref, bl_ref, out_ref):
    num = pn_ref[0, :N, :] + pn_ref[1, :N, :]
    den = pd_ref[0, :N, :] + pd_ref[1, :N, :]
    xl = num / (den + 1e-16) + b_ref[...]
    xl = jnp.where(xl > 0, xl, jnp.exp(xl) - 1.0)
    out_ref[...] = jnp.dot(xl, wl_ref[...],
                           preferred_element_type=jnp.float32) + bl_ref[...]


_final = pl.pallas_call(
    _final_body,
    out_shape=jax.ShapeDtypeStruct((N, 1), jnp.float32),
)


def kernel(x, edge_index, W_src1, W_dst1, a_src1, a_dst1, b1,
           W_src2, W_dst2, a_src2, a_dst2, b2, W_lin, b_lin):
    src = edge_index[0]
    dst = edge_index[1]
    pad = E_PAD + ESTEP - E
    srcp = jnp.concatenate([src, jnp.zeros((pad,), jnp.int32)])
    dstp = jnp.concatenate([dst, jnp.full((pad,), N, jnp.int32)])
    z1 = jnp.zeros((ZROWS, H), jnp.float32)
    z2 = jnp.zeros((DROWS, 128), jnp.float32)

    h1, as1, ad1, _m1 = _prep(x, W_src1, W_dst1,
                              a_src1.reshape(H, 1), a_dst1.reshape(H, 1))
    pn1, pd1 = _edge_pass(h1, as1.reshape(N), ad1.reshape(N),
                          srcp, dstp, z1, z2)
    pd1 = pd1.reshape(NC, N_ACC, 1)
    h2, as2, ad2, _m2 = _combine_prep(pn1, pd1, b1.reshape(1, H),
                                      W_src2, W_dst2,
                                      a_src2.reshape(H, 1), a_dst2.reshape(H, 1))
    pn2, pd2 = _edge_pass(h2, as2.reshape(N), ad2.reshape(N),
                          srcp, dstp, z1, z2)
    pd2 = pd2.reshape(NC, N_ACC, 1)
    out = _final(pn2, pd2, b2.reshape(1, H), W_lin, b_lin.reshape(1, 1))
    return out.reshape(N)
